# Initial kernel scaffold; baseline (speedup 1.0000x reference)
import jax
import jax.numpy as jnp
from jax.experimental import pallas as pl


def kernel(epes_stat_flow, epes_dyn_flow, moving_mask, dynamicness_scores, moving_average_importance, training):
    raise NotImplementedError("write your pallas kernel here")


# speedup vs baseline: 18.7029x; 18.7029x over previous
import dataclasses
import functools

import jax
import jax.numpy as jnp
import numpy as np
from jax import lax
from jax.experimental import pallas as pl
from jax.experimental.pallas import tpu as pltpu
from jax.experimental.pallas import tpu_sc as plsc

N = 4000000
RES = 100000
ROWS = 784
HIST_PAD = ROWS * 128
NW = 32

W_IMP = float(np.float32(1.0) / np.float32(1e8))
SCALE = float(np.float32(RES) / np.float32(1.0))

_TOTAL = 100000000 + 100000000
_AVG_PTS = _TOTAL / 1000
_UW = 1.0 / min(2.0 * _TOTAL, 5000.0 * _AVG_PTS)
CUW = float(np.float32((1.0 - _UW) ** float(N)))

CNT_HI = 125008
CNT_LO = 124992
CH = 4096
N_FULL = 30
REM = 2112


def _sc_hist_body(stat_hbm, dyn_hbm, score_hbm, out_hbm,
                  hist_v, stat_v, dyn_v, score_v):
    wid = lax.axis_index("s") * 2 + lax.axis_index("c")
    is_hi = wid < 16
    base = jnp.where(is_hi, wid * CNT_HI,
                     16 * CNT_HI + (wid - 16) * CNT_LO)

    zero = jnp.zeros((16,), jnp.float32)

    @pl.loop(0, HIST_PAD // 16)
    def _(i):
        hist_v[pl.ds(i * 16, 16)] = zero

    def process(nvreg):
        def body(j):
            a = stat_v[pl.ds(j * 16, 16)]
            b = dyn_v[pl.ds(j * 16, 16)]
            s = score_v[pl.ds(j * 16, 16)]
            val = (a - b) * np.float32(W_IMP)
            idx = lax.convert_element_type(s * np.float32(SCALE), jnp.int32)
            idx = jnp.minimum(jnp.maximum(idx, 0), RES - 1)
            plsc.addupdate_scatter(hist_v, [idx], val)
        return body

    @pl.loop(0, N_FULL)
    def _(c):
        off = base + c * CH
        pltpu.sync_copy(stat_hbm.at[pl.ds(off, CH)], stat_v.at[pl.ds(0, CH)])
        pltpu.sync_copy(dyn_hbm.at[pl.ds(off, CH)], dyn_v.at[pl.ds(0, CH)])
        pltpu.sync_copy(score_hbm.at[pl.ds(off, CH)], score_v.at[pl.ds(0, CH)])
        pl.loop(0, CH // 16)(process(CH // 16))

    off = base + N_FULL * CH
    pltpu.sync_copy(stat_hbm.at[pl.ds(off, REM)], stat_v.at[pl.ds(0, REM)])
    pltpu.sync_copy(dyn_hbm.at[pl.ds(off, REM)], dyn_v.at[pl.ds(0, REM)])
    pltpu.sync_copy(score_hbm.at[pl.ds(off, REM)], score_v.at[pl.ds(0, REM)])
    pl.loop(0, REM // 16)(process(REM // 16))

    @pl.when(is_hi)
    def _():
        off2 = base + N_FULL * CH + REM
        pltpu.sync_copy(stat_hbm.at[pl.ds(off2, 16)], stat_v.at[pl.ds(0, 16)])
        pltpu.sync_copy(dyn_hbm.at[pl.ds(off2, 16)], dyn_v.at[pl.ds(0, 16)])
        pltpu.sync_copy(score_hbm.at[pl.ds(off2, 16)], score_v.at[pl.ds(0, 16)])
        process(1)(0)

    pltpu.sync_copy(hist_v, out_hbm.at[wid])


@jax.jit
def _sc_hist(stat, dyn, score):
    mesh = plsc.VectorSubcoreMesh(core_axis_name="c", subcore_axis_name="s")
    cp = pltpu.CompilerParams()
    if "needs_layout_passes" in pltpu.CompilerParams.__dataclass_fields__:
        cp = dataclasses.replace(cp, needs_layout_passes=False)
    f = pl.kernel(
        _sc_hist_body,
        out_type=jax.ShapeDtypeStruct((NW, HIST_PAD), jnp.float32),
        mesh=mesh,
        scratch_types=[
            pltpu.VMEM((HIST_PAD,), jnp.float32),
            pltpu.VMEM((CH,), jnp.float32),
            pltpu.VMEM((CH,), jnp.float32),
            pltpu.VMEM((CH,), jnp.float32),
        ],
        compiler_params=cp,
    )
    return f(stat, dyn, score)


def _tc_post_body(ph_ref, mai_ref, out_ref):
    h = jnp.sum(ph_ref[...], axis=0)
    mai = mai_ref[...] * np.float32(CUW) + np.float32(1.0 - CUW) * h

    ii = lax.broadcasted_iota(jnp.int32, (128, 128), 0)
    jj = lax.broadcasted_iota(jnp.int32, (128, 128), 1)
    upper = (ii <= jj).astype(jnp.float32)
    w = lax.dot_general(mai, upper, (((1,), (0,)), ((), ())),
                        preferred_element_type=jnp.float32,
                        precision=lax.Precision.HIGHEST)

    rr = lax.broadcasted_iota(jnp.int32, (ROWS, ROWS), 0)
    cc = lax.broadcasted_iota(jnp.int32, (ROWS, ROWS), 1)
    lstrict = (cc < rr).astype(jnp.float32)
    s_b = jnp.broadcast_to(w[:, 127:128], (ROWS, 128))
    p = lax.dot_general(lstrict, s_b, (((1,), (0,)), ((), ())),
                        preferred_element_type=jnp.float32,
                        precision=lax.Precision.HIGHEST)
    c = w + p

    r2 = lax.broadcasted_iota(jnp.int32, (ROWS, 128), 0)
    l2 = lax.broadcasted_iota(jnp.int32, (ROWS, 128), 1)
    k = r2 * 128 + l2
    valid = k < RES
    cv = jnp.where(valid, c, jnp.float32(jnp.inf))
    best = jnp.minimum(jnp.min(cv), jnp.float32(0.0))
    eq = cv == best
    cnt = jnp.sum(eq.astype(jnp.float32)) + (best == 0.0).astype(jnp.float32)
    idxsum = jnp.sum(jnp.where(eq, (k + 1).astype(jnp.float32), jnp.float32(0.0)))
    avg = idxsum / cnt
    out_ref[...] = jnp.broadcast_to(avg * np.float32(1.0) / np.float32(RES), (1, 1))


@jax.jit
def _tc_post(part3, mai_pad):
    return pl.pallas_call(
        _tc_post_body,
        out_shape=jax.ShapeDtypeStruct((1, 1), jnp.float32),
    )(part3, mai_pad)


def kernel(epes_stat_flow, epes_dyn_flow, moving_mask, dynamicness_scores,
           moving_average_importance, training=True):
    part = _sc_hist(epes_stat_flow, epes_dyn_flow, dynamicness_scores)
    part3 = part.reshape(NW, ROWS, 128)
    mai_pad = jnp.pad(moving_average_importance, (0, HIST_PAD - RES)).reshape(ROWS, 128)
    out = _tc_post(part3, mai_pad)
    return out[0, 0]


# speedup vs baseline: 33.4479x; 1.7884x over previous
import dataclasses
import functools

import jax
import jax.numpy as jnp
import numpy as np
from jax import lax
from jax.experimental import pallas as pl
from jax.experimental.pallas import tpu as pltpu
from jax.experimental.pallas import tpu_sc as plsc

N = 4000000
RES = 100000
ROWS = 784
HIST_PAD = ROWS * 128
NW = 32

W_IMP = float(np.float32(1.0) / np.float32(1e8))
SCALE = float(np.float32(RES) / np.float32(1.0))

_TOTAL = 100000000 + 100000000
_AVG_PTS = _TOTAL / 1000
_UW = 1.0 / min(2.0 * _TOTAL, 5000.0 * _AVG_PTS)
CUW = float(np.float32((1.0 - _UW) ** float(N)))

CNT_HI = 125008
CNT_LO = 124992
CH = 4096
N_FULL = 30
REM = 2112


def _sc_hist_body(stat_hbm, dyn_hbm, score_hbm, out_hbm,
                  hist_v, stat_v, dyn_v, score_v, sem0, sem1):
    wid = lax.axis_index("s") * 2 + lax.axis_index("c")
    is_hi = wid < 16
    base = jnp.where(is_hi, wid * CNT_HI,
                     16 * CNT_HI + (wid - 16) * CNT_LO)

    zero = jnp.zeros((16,), jnp.float32)

    @pl.loop(0, HIST_PAD // (16 * 8))
    def _(i):
        for u in range(8):
            hist_v[pl.ds(i * 128 + u * 16, 16)] = zero

    def scatter_vreg(a, b, s):
        val = (a - b) * np.float32(W_IMP)
        idx = lax.convert_element_type(s * np.float32(SCALE), jnp.int32)
        idx = jnp.minimum(jnp.maximum(idx, 0), RES - 1)
        plsc.addupdate_scatter(hist_v, [idx], val)

    def compute(slot, nvreg, unroll):
        @pl.loop(0, nvreg // unroll)
        def _(j):
            for u in range(unroll):
                o = j * (unroll * 16) + u * 16
                scatter_vreg(stat_v[pl.ds(slot * CH + o, 16)],
                             dyn_v[pl.ds(slot * CH + o, 16)],
                             score_v[pl.ds(slot * CH + o, 16)])

    def copies(slot, c, sem):
        off = base + c * CH
        return [
            pltpu.make_async_copy(stat_hbm.at[pl.ds(off, CH)],
                                  stat_v.at[pl.ds(slot * CH, CH)], sem),
            pltpu.make_async_copy(dyn_hbm.at[pl.ds(off, CH)],
                                  dyn_v.at[pl.ds(slot * CH, CH)], sem),
            pltpu.make_async_copy(score_hbm.at[pl.ds(off, CH)],
                                  score_v.at[pl.ds(slot * CH, CH)], sem),
        ]

    def start(slot, c, sem):
        for cp in copies(slot, c, sem):
            cp.start()

    def wait(slot, c, sem):
        for cp in copies(slot, c, sem):
            cp.wait()

    start(0, 0, sem0)
    start(1, 1, sem1)

    @pl.loop(0, N_FULL // 2)
    def _(i):
        wait(0, 2 * i, sem0)
        compute(0, CH // 16, 8)

        @pl.when(i < N_FULL // 2 - 1)
        def _():
            start(0, 2 * i + 2, sem0)

        wait(1, 2 * i + 1, sem1)
        compute(1, CH // 16, 8)

        @pl.when(i < N_FULL // 2 - 1)
        def _():
            start(1, 2 * i + 3, sem1)

    off = base + N_FULL * CH
    pltpu.sync_copy(stat_hbm.at[pl.ds(off, REM)], stat_v.at[pl.ds(0, REM)])
    pltpu.sync_copy(dyn_hbm.at[pl.ds(off, REM)], dyn_v.at[pl.ds(0, REM)])
    pltpu.sync_copy(score_hbm.at[pl.ds(off, REM)], score_v.at[pl.ds(0, REM)])
    compute(0, REM // 16, 4)

    @pl.when(is_hi)
    def _():
        off2 = base + N_FULL * CH + REM
        pltpu.sync_copy(stat_hbm.at[pl.ds(off2, 16)], stat_v.at[pl.ds(0, 16)])
        pltpu.sync_copy(dyn_hbm.at[pl.ds(off2, 16)], dyn_v.at[pl.ds(0, 16)])
        pltpu.sync_copy(score_hbm.at[pl.ds(off2, 16)], score_v.at[pl.ds(0, 16)])
        scatter_vreg(stat_v[pl.ds(0, 16)], dyn_v[pl.ds(0, 16)],
                     score_v[pl.ds(0, 16)])

    pltpu.sync_copy(hist_v, out_hbm.at[wid])


@jax.jit
def _sc_hist(stat, dyn, score):
    mesh = plsc.VectorSubcoreMesh(core_axis_name="c", subcore_axis_name="s")
    cp = pltpu.CompilerParams()
    if "needs_layout_passes" in pltpu.CompilerParams.__dataclass_fields__:
        cp = dataclasses.replace(cp, needs_layout_passes=False)
    f = pl.kernel(
        _sc_hist_body,
        out_type=jax.ShapeDtypeStruct((NW, HIST_PAD), jnp.float32),
        mesh=mesh,
        scratch_types=[
            pltpu.VMEM((HIST_PAD,), jnp.float32),
            pltpu.VMEM((2 * CH,), jnp.float32),
            pltpu.VMEM((2 * CH,), jnp.float32),
            pltpu.VMEM((2 * CH,), jnp.float32),
            pltpu.SemaphoreType.DMA,
            pltpu.SemaphoreType.DMA,
        ],
        compiler_params=cp,
    )
    return f(stat, dyn, score)


def _tc_post_body(ph_ref, mai_ref, out_ref):
    h = jnp.sum(ph_ref[...].reshape(NW, ROWS, 128), axis=0)
    mai = mai_ref[...] * np.float32(CUW) + np.float32(1.0 - CUW) * h

    ii = lax.broadcasted_iota(jnp.int32, (128, 128), 0)
    jj = lax.broadcasted_iota(jnp.int32, (128, 128), 1)
    upper = (ii <= jj).astype(jnp.float32)
    w = lax.dot_general(mai, upper, (((1,), (0,)), ((), ())),
                        preferred_element_type=jnp.float32,
                        precision=lax.Precision.HIGHEST)

    rr = lax.broadcasted_iota(jnp.int32, (ROWS, ROWS), 0)
    cc = lax.broadcasted_iota(jnp.int32, (ROWS, ROWS), 1)
    lstrict = (cc < rr).astype(jnp.float32)
    s_b = jnp.broadcast_to(w[:, 127:128], (ROWS, 128))
    p = lax.dot_general(lstrict, s_b, (((1,), (0,)), ((), ())),
                        preferred_element_type=jnp.float32,
                        precision=lax.Precision.HIGHEST)
    c = w + p

    r2 = lax.broadcasted_iota(jnp.int32, (ROWS, 128), 0)
    l2 = lax.broadcasted_iota(jnp.int32, (ROWS, 128), 1)
    k = r2 * 128 + l2
    valid = k < RES
    cv = jnp.where(valid, c, jnp.float32(jnp.inf))
    best = jnp.minimum(jnp.min(cv), jnp.float32(0.0))
    eq = cv == best
    cnt = jnp.sum(eq.astype(jnp.float32)) + (best == 0.0).astype(jnp.float32)
    idxsum = jnp.sum(jnp.where(eq, (k + 1).astype(jnp.float32), jnp.float32(0.0)))
    avg = idxsum / cnt
    out_ref[...] = jnp.broadcast_to(avg * np.float32(1.0) / np.float32(RES), (1, 1))


@jax.jit
def _tc_post(part, mai_pad):
    return pl.pallas_call(
        _tc_post_body,
        out_shape=jax.ShapeDtypeStruct((1, 1), jnp.float32),
    )(part, mai_pad)


def kernel(epes_stat_flow, epes_dyn_flow, moving_mask, dynamicness_scores,
           moving_average_importance, training=True):
    part = _sc_hist(epes_stat_flow, epes_dyn_flow, dynamicness_scores)
    mai_pad = jnp.pad(moving_average_importance, (0, HIST_PAD - RES)).reshape(ROWS, 128)
    out = _tc_post(part, mai_pad)
    return out[0, 0]


# speedup vs baseline: 61.5237x; 1.8394x over previous
import dataclasses
import functools

import jax
import jax.numpy as jnp
import numpy as np
from jax import lax
from jax.experimental import pallas as pl
from jax.experimental.pallas import tpu as pltpu
from jax.experimental.pallas import tpu_sc as plsc

N = 4000000
RES = 100000
ROWS = 784
HIST_PAD = ROWS * 128
NW = 32

W_IMP = float(np.float32(1.0) / np.float32(1e8))
SCALE = float(np.float32(RES) / np.float32(1.0))

_TOTAL = 100000000 + 100000000
_AVG_PTS = _TOTAL / 1000
_UW = 1.0 / min(2.0 * _TOTAL, 5000.0 * _AVG_PTS)
CUW = float(np.float32((1.0 - _UW) ** float(N)))

CNT_HI = 125008
CNT_LO = 124992
CH = 4096
N_FULL = 30
REM = 2112


def _sc_hist_body(stat_hbm, dyn_hbm, score_hbm, out_hbm,
                  hist_v, stat_v, dyn_v, score_v, sem0, sem1):
    wid = lax.axis_index("s") * 2 + lax.axis_index("c")
    is_hi = wid < 16
    base = jnp.where(is_hi, wid * CNT_HI,
                     16 * CNT_HI + (wid - 16) * CNT_LO)

    zero = jnp.zeros((16,), jnp.float32)

    @pl.loop(0, HIST_PAD // (16 * 8))
    def _(i):
        for u in range(8):
            hist_v[pl.ds(i * 128 + u * 16, 16)] = zero

    def scatter_vreg(a, b, s):
        val = (a - b) * np.float32(W_IMP)
        idx = lax.convert_element_type(s * np.float32(SCALE), jnp.int32)
        idx = jnp.minimum(jnp.maximum(idx, 0), RES - 1)
        plsc.addupdate_scatter(hist_v, [idx], val)

    def compute(slot, nvreg, unroll):
        @plsc.parallel_loop(0, nvreg * 16, 16, unroll=unroll)
        def _(o):
            scatter_vreg(stat_v[pl.ds(slot * CH + o, 16)],
                         dyn_v[pl.ds(slot * CH + o, 16)],
                         score_v[pl.ds(slot * CH + o, 16)])

    def copies(slot, c, sem):
        off = base + c * CH
        return [
            pltpu.make_async_copy(stat_hbm.at[pl.ds(off, CH)],
                                  stat_v.at[pl.ds(slot * CH, CH)], sem),
            pltpu.make_async_copy(dyn_hbm.at[pl.ds(off, CH)],
                                  dyn_v.at[pl.ds(slot * CH, CH)], sem),
            pltpu.make_async_copy(score_hbm.at[pl.ds(off, CH)],
                                  score_v.at[pl.ds(slot * CH, CH)], sem),
        ]

    def start(slot, c, sem):
        for cp in copies(slot, c, sem):
            cp.start()

    def wait(slot, c, sem):
        for cp in copies(slot, c, sem):
            cp.wait()

    start(0, 0, sem0)
    start(1, 1, sem1)

    @pl.loop(0, N_FULL // 2)
    def _(i):
        wait(0, 2 * i, sem0)
        compute(0, CH // 16, 8)

        @pl.when(i < N_FULL // 2 - 1)
        def _():
            start(0, 2 * i + 2, sem0)

        wait(1, 2 * i + 1, sem1)
        compute(1, CH // 16, 8)

        @pl.when(i < N_FULL // 2 - 1)
        def _():
            start(1, 2 * i + 3, sem1)

    off = base + N_FULL * CH
    pltpu.sync_copy(stat_hbm.at[pl.ds(off, REM)], stat_v.at[pl.ds(0, REM)])
    pltpu.sync_copy(dyn_hbm.at[pl.ds(off, REM)], dyn_v.at[pl.ds(0, REM)])
    pltpu.sync_copy(score_hbm.at[pl.ds(off, REM)], score_v.at[pl.ds(0, REM)])
    compute(0, REM // 16, 4)

    @pl.when(is_hi)
    def _():
        off2 = base + N_FULL * CH + REM
        pltpu.sync_copy(stat_hbm.at[pl.ds(off2, 16)], stat_v.at[pl.ds(0, 16)])
        pltpu.sync_copy(dyn_hbm.at[pl.ds(off2, 16)], dyn_v.at[pl.ds(0, 16)])
        pltpu.sync_copy(score_hbm.at[pl.ds(off2, 16)], score_v.at[pl.ds(0, 16)])
        scatter_vreg(stat_v[pl.ds(0, 16)], dyn_v[pl.ds(0, 16)],
                     score_v[pl.ds(0, 16)])

    pltpu.sync_copy(hist_v, out_hbm.at[wid])


@jax.jit
def _sc_hist(stat, dyn, score):
    mesh = plsc.VectorSubcoreMesh(core_axis_name="c", subcore_axis_name="s")
    cp = pltpu.CompilerParams()
    if "needs_layout_passes" in pltpu.CompilerParams.__dataclass_fields__:
        cp = dataclasses.replace(cp, needs_layout_passes=False)
    f = pl.kernel(
        _sc_hist_body,
        out_type=jax.ShapeDtypeStruct((NW, HIST_PAD), jnp.float32),
        mesh=mesh,
        scratch_types=[
            pltpu.VMEM((HIST_PAD,), jnp.float32),
            pltpu.VMEM((2 * CH,), jnp.float32),
            pltpu.VMEM((2 * CH,), jnp.float32),
            pltpu.VMEM((2 * CH,), jnp.float32),
            pltpu.SemaphoreType.DMA,
            pltpu.SemaphoreType.DMA,
        ],
        compiler_params=cp,
    )
    return f(stat, dyn, score)


def _tc_post_body(ph_ref, mai_ref, out_ref):
    h = jnp.sum(ph_ref[...].reshape(NW, ROWS, 128), axis=0)
    mai = mai_ref[...] * np.float32(CUW) + np.float32(1.0 - CUW) * h

    ii = lax.broadcasted_iota(jnp.int32, (128, 128), 0)
    jj = lax.broadcasted_iota(jnp.int32, (128, 128), 1)
    upper = (ii <= jj).astype(jnp.float32)
    w = lax.dot_general(mai, upper, (((1,), (0,)), ((), ())),
                        preferred_element_type=jnp.float32,
                        precision=lax.Precision.HIGHEST)

    rr = lax.broadcasted_iota(jnp.int32, (ROWS, ROWS), 0)
    cc = lax.broadcasted_iota(jnp.int32, (ROWS, ROWS), 1)
    lstrict = (cc < rr).astype(jnp.float32)
    s_b = jnp.broadcast_to(w[:, 127:128], (ROWS, 128))
    p = lax.dot_general(lstrict, s_b, (((1,), (0,)), ((), ())),
                        preferred_element_type=jnp.float32,
                        precision=lax.Precision.HIGHEST)
    c = w + p

    r2 = lax.broadcasted_iota(jnp.int32, (ROWS, 128), 0)
    l2 = lax.broadcasted_iota(jnp.int32, (ROWS, 128), 1)
    k = r2 * 128 + l2
    valid = k < RES
    cv = jnp.where(valid, c, jnp.float32(jnp.inf))
    best = jnp.minimum(jnp.min(cv), jnp.float32(0.0))
    eq = cv == best
    cnt = jnp.sum(eq.astype(jnp.float32)) + (best == 0.0).astype(jnp.float32)
    idxsum = jnp.sum(jnp.where(eq, (k + 1).astype(jnp.float32), jnp.float32(0.0)))
    avg = idxsum / cnt
    out_ref[...] = jnp.broadcast_to(avg * np.float32(1.0) / np.float32(RES), (1, 1))


@jax.jit
def _tc_post(part, mai_pad):
    return pl.pallas_call(
        _tc_post_body,
        out_shape=jax.ShapeDtypeStruct((1, 1), jnp.float32),
    )(part, mai_pad)


def kernel(epes_stat_flow, epes_dyn_flow, moving_mask, dynamicness_scores,
           moving_average_importance, training=True):
    part = _sc_hist(epes_stat_flow, epes_dyn_flow, dynamicness_scores)
    mai_pad = jnp.pad(moving_average_importance, (0, HIST_PAD - RES)).reshape(ROWS, 128)
    out = _tc_post(part, mai_pad)
    return out[0, 0]


# speedup vs baseline: 61.5877x; 1.0010x over previous
import dataclasses
import functools

import jax
import jax.numpy as jnp
import numpy as np
from jax import lax
from jax.experimental import pallas as pl
from jax.experimental.pallas import tpu as pltpu
from jax.experimental.pallas import tpu_sc as plsc

N = 4000000
RES = 100000
ROWS = 784
HIST_PAD = ROWS * 128
NW = 32

W_IMP = float(np.float32(1.0) / np.float32(1e8))
SCALE = float(np.float32(RES) / np.float32(1.0))

_TOTAL = 100000000 + 100000000
_AVG_PTS = _TOTAL / 1000
_UW = 1.0 / min(2.0 * _TOTAL, 5000.0 * _AVG_PTS)
CUW = float(np.float32((1.0 - _UW) ** float(N)))

CNT_HI = 125008
CNT_LO = 124992
CH = 4096
N_FULL = 30
REM = 2112


def _sc_hist_body(stat_hbm, dyn_hbm, score_hbm, out_hbm,
                  hist_v, stat_v, dyn_v, score_v, sem0, sem1):
    wid = lax.axis_index("s") * 2 + lax.axis_index("c")
    is_hi = wid < 16
    base = jnp.where(is_hi, wid * CNT_HI,
                     16 * CNT_HI + (wid - 16) * CNT_LO)

    zero = jnp.zeros((16,), jnp.float32)

    @pl.loop(0, HIST_PAD // (16 * 8))
    def _(i):
        for u in range(8):
            hist_v[pl.ds(i * 128 + u * 16, 16)] = zero

    def scatter_vreg(a, b, s):
        val = a - b
        idx = lax.convert_element_type(s * np.float32(SCALE), jnp.int32)
        idx = jnp.minimum(jnp.maximum(idx, 0), RES - 1)
        plsc.addupdate_scatter(hist_v, [idx], val)

    def compute(slot, nvreg, unroll):
        @plsc.parallel_loop(0, nvreg * 16, 16, unroll=unroll)
        def _(o):
            scatter_vreg(stat_v[pl.ds(slot * CH + o, 16)],
                         dyn_v[pl.ds(slot * CH + o, 16)],
                         score_v[pl.ds(slot * CH + o, 16)])

    def copies(slot, c, sem):
        off = base + c * CH
        return [
            pltpu.make_async_copy(stat_hbm.at[pl.ds(off, CH)],
                                  stat_v.at[pl.ds(slot * CH, CH)], sem),
            pltpu.make_async_copy(dyn_hbm.at[pl.ds(off, CH)],
                                  dyn_v.at[pl.ds(slot * CH, CH)], sem),
            pltpu.make_async_copy(score_hbm.at[pl.ds(off, CH)],
                                  score_v.at[pl.ds(slot * CH, CH)], sem),
        ]

    def start(slot, c, sem):
        for cp in copies(slot, c, sem):
            cp.start()

    def wait(slot, c, sem):
        for cp in copies(slot, c, sem):
            cp.wait()

    start(0, 0, sem0)
    start(1, 1, sem1)

    @pl.loop(0, N_FULL // 2)
    def _(i):
        wait(0, 2 * i, sem0)
        compute(0, CH // 16, 16)

        @pl.when(i < N_FULL // 2 - 1)
        def _():
            start(0, 2 * i + 2, sem0)

        wait(1, 2 * i + 1, sem1)
        compute(1, CH // 16, 16)

        @pl.when(i < N_FULL // 2 - 1)
        def _():
            start(1, 2 * i + 3, sem1)

    off = base + N_FULL * CH
    pltpu.sync_copy(stat_hbm.at[pl.ds(off, REM)], stat_v.at[pl.ds(0, REM)])
    pltpu.sync_copy(dyn_hbm.at[pl.ds(off, REM)], dyn_v.at[pl.ds(0, REM)])
    pltpu.sync_copy(score_hbm.at[pl.ds(off, REM)], score_v.at[pl.ds(0, REM)])
    compute(0, REM // 16, 4)

    @pl.when(is_hi)
    def _():
        off2 = base + N_FULL * CH + REM
        pltpu.sync_copy(stat_hbm.at[pl.ds(off2, 16)], stat_v.at[pl.ds(0, 16)])
        pltpu.sync_copy(dyn_hbm.at[pl.ds(off2, 16)], dyn_v.at[pl.ds(0, 16)])
        pltpu.sync_copy(score_hbm.at[pl.ds(off2, 16)], score_v.at[pl.ds(0, 16)])
        scatter_vreg(stat_v[pl.ds(0, 16)], dyn_v[pl.ds(0, 16)],
                     score_v[pl.ds(0, 16)])

    pltpu.sync_copy(hist_v, out_hbm.at[wid])


@jax.jit
def _sc_hist(stat, dyn, score):
    mesh = plsc.VectorSubcoreMesh(core_axis_name="c", subcore_axis_name="s")
    cp = pltpu.CompilerParams()
    if "needs_layout_passes" in pltpu.CompilerParams.__dataclass_fields__:
        cp = dataclasses.replace(cp, needs_layout_passes=False)
    f = pl.kernel(
        _sc_hist_body,
        out_type=jax.ShapeDtypeStruct((NW, HIST_PAD), jnp.float32),
        mesh=mesh,
        scratch_types=[
            pltpu.VMEM((HIST_PAD,), jnp.float32),
            pltpu.VMEM((2 * CH,), jnp.float32),
            pltpu.VMEM((2 * CH,), jnp.float32),
            pltpu.VMEM((2 * CH,), jnp.float32),
            pltpu.SemaphoreType.DMA,
            pltpu.SemaphoreType.DMA,
        ],
        compiler_params=cp,
    )
    return f(stat, dyn, score)


def _tc_post_body(ph_ref, mai_ref, out_ref):
    h = jnp.sum(ph_ref[...].reshape(NW, ROWS, 128), axis=0)
    mai = mai_ref[...] * np.float32(CUW) + (np.float32(1.0 - CUW) * np.float32(W_IMP)) * h

    ii = lax.broadcasted_iota(jnp.int32, (128, 128), 0)
    jj = lax.broadcasted_iota(jnp.int32, (128, 128), 1)
    upper = (ii <= jj).astype(jnp.float32)
    w = lax.dot_general(mai, upper, (((1,), (0,)), ((), ())),
                        preferred_element_type=jnp.float32,
                        precision=lax.Precision.HIGHEST)

    rr = lax.broadcasted_iota(jnp.int32, (ROWS, ROWS), 0)
    cc = lax.broadcasted_iota(jnp.int32, (ROWS, ROWS), 1)
    lstrict = (cc < rr).astype(jnp.float32)
    s_b = jnp.broadcast_to(w[:, 127:128], (ROWS, 128))
    p = lax.dot_general(lstrict, s_b, (((1,), (0,)), ((), ())),
                        preferred_element_type=jnp.float32,
                        precision=lax.Precision.HIGHEST)
    c = w + p

    r2 = lax.broadcasted_iota(jnp.int32, (ROWS, 128), 0)
    l2 = lax.broadcasted_iota(jnp.int32, (ROWS, 128), 1)
    k = r2 * 128 + l2
    valid = k < RES
    cv = jnp.where(valid, c, jnp.float32(jnp.inf))
    best = jnp.minimum(jnp.min(cv), jnp.float32(0.0))
    eq = cv == best
    cnt = jnp.sum(eq.astype(jnp.float32)) + (best == 0.0).astype(jnp.float32)
    idxsum = jnp.sum(jnp.where(eq, (k + 1).astype(jnp.float32), jnp.float32(0.0)))
    avg = idxsum / cnt
    out_ref[...] = jnp.broadcast_to(avg * np.float32(1.0) / np.float32(RES), (1, 1))


@jax.jit
def _tc_post(part, mai_pad):
    return pl.pallas_call(
        _tc_post_body,
        out_shape=jax.ShapeDtypeStruct((1, 1), jnp.float32),
    )(part, mai_pad)


def kernel(epes_stat_flow, epes_dyn_flow, moving_mask, dynamicness_scores,
           moving_average_importance, training=True):
    part = _sc_hist(epes_stat_flow, epes_dyn_flow, dynamicness_scores)
    mai_pad = jnp.pad(moving_average_importance, (0, HIST_PAD - RES)).reshape(ROWS, 128)
    out = _tc_post(part, mai_pad)
    return out[0, 0]
